# Initial kernel scaffold; baseline (speedup 1.0000x reference)
import jax
import jax.numpy as jnp
from jax.experimental import pallas as pl


def kernel(x, edge_index, A1, Ab1, mu1, sigma1, W1, b1, A2, Ab2, mu2, sigma2, W2, b2, A3, Ab3, mu3, sigma3, W3, b3):
    raise NotImplementedError("write your pallas kernel here")


# speedup vs baseline: 6.3205x; 6.3205x over previous
import functools

import jax
import jax.numpy as jnp
from jax import lax
from jax.experimental import pallas as pl
from jax.experimental.pallas import tpu as pltpu
from jax.experimental.pallas import tpu_sc as plsc

N = 10000
E = 320000
K = 4
EMB = 16

NC = 2
NS = 16
NW = NC * NS
EPW = E // NW
C = 80
NCHUNK = EPW // C
RPT = N // NS
ZROWS = 125

_mesh = plsc.VectorSubcoreMesh(core_axis_name="c", subcore_axis_name="s")
_sc_params = pltpu.CompilerParams(use_tc_tiling_on_sc=False)


def _mm_body(x_ref, w_ref, o_ref):
    o_ref[...] = jnp.dot(x_ref[...], w_ref[...],
                         preferred_element_type=jnp.float32)


def _matmul(x, w, bn):
    n, kin = x.shape
    ko = w.shape[1]
    return pl.pallas_call(
        _mm_body,
        grid=(n // bn,),
        in_specs=[pl.BlockSpec((bn, kin), lambda i: (i, 0)),
                  pl.BlockSpec((kin, ko), lambda i: (0, 0))],
        out_specs=pl.BlockSpec((bn, ko), lambda i: (i, 0)),
        out_shape=jax.ShapeDtypeStruct((n, ko), jnp.float32),
    )(x, w)


def _wts_body(t_ref, m2_ref, m1_ref, c_ref, o_ref):
    t = t_ref[...]
    h = (jnp.dot(t * t, m2_ref[...], preferred_element_type=jnp.float32)
         + jnp.dot(t, m1_ref[...], preferred_element_type=jnp.float32)
         + c_ref[...])
    o_ref[...] = jnp.exp(h)


def _edge_weights(t, m2, m1, c, bn=2000):
    n = t.shape[0]
    return pl.pallas_call(
        _wts_body,
        grid=(n // bn,),
        in_specs=[pl.BlockSpec((bn, 48), lambda i: (i, 0)),
                  pl.BlockSpec((48, 16), lambda i: (0, 0)),
                  pl.BlockSpec((48, 16), lambda i: (0, 0)),
                  pl.BlockSpec((1, 16), lambda i: (0, 0))],
        out_specs=pl.BlockSpec((bn, 16), lambda i: (i, 0)),
        out_shape=jax.ShapeDtypeStruct((n, 16), jnp.float32),
    )(t, m2, m1, c)


def _diff_body(pcat_hbm, src_hbm, dst_hbm, out_hbm, idx_s, idx_d, ps, pd, sem):
    cid = lax.axis_index("c")
    sid = lax.axis_index("s")
    wid = cid * NS + sid
    base = wid * EPW

    def chunk(g, _):
        off = base + g * C
        pltpu.sync_copy(src_hbm.at[pl.ds(off, C)], idx_s)
        pltpu.sync_copy(dst_hbm.at[pl.ds(off, C)], idx_d)
        pltpu.async_copy(pcat_hbm.at[idx_s], ps, sem).wait()
        pltpu.async_copy(pcat_hbm.at[idx_d], pd, sem).wait()

        def edge(e, _):
            for j in range(3):
                sl = pl.ds(j * 16, 16)
                ps[e, sl] = ps[e, sl] - pd[e, sl]
            return _

        lax.fori_loop(0, C, edge, None)
        pltpu.sync_copy(ps, out_hbm.at[pl.ds(off, C)])
        return _

    lax.fori_loop(0, NCHUNK, chunk, None)


_diff_kernel = pl.kernel(
    _diff_body,
    out_type=jax.ShapeDtypeStruct((E, 48), jnp.float32),
    mesh=_mesh,
    compiler_params=_sc_params,
    scratch_types=[
        pltpu.VMEM((C,), jnp.int32),
        pltpu.VMEM((C,), jnp.int32),
        pltpu.VMEM((C, 48), jnp.float32),
        pltpu.VMEM((C, 48), jnp.float32),
        pltpu.SemaphoreType.DMA,
    ],
)


def _make_agg_body(out_dim, layer):
    ko = K * out_dim
    nj = out_dim // 16
    col = 4 * layer

    def body(y_hbm, w_hbm, src_hbm, dst_hbm, part_hbm,
             idx_s, idx_d, wv, rows, msg, zbuf, acc_sh, sem):
        cid = lax.axis_index("c")
        sid = lax.axis_index("s")
        wid = cid * NS + sid
        base = wid * EPW

        def zrow(i, _):
            for j in range(nj):
                zbuf[i, pl.ds(j * 16, 16)] = jnp.zeros((16,), jnp.float32)
            return _

        lax.fori_loop(0, ZROWS, zrow, None)
        for r in range(RPT // ZROWS):
            pltpu.sync_copy(zbuf, acc_sh.at[pl.ds(sid * RPT + r * ZROWS, ZROWS)])
        plsc.subcore_barrier()

        def chunk(g, _):
            off = base + g * C
            pltpu.sync_copy(src_hbm.at[pl.ds(off, C)], idx_s)
            pltpu.sync_copy(dst_hbm.at[pl.ds(off, C)], idx_d)
            pltpu.sync_copy(w_hbm.at[pl.ds(off, C)], wv)
            pltpu.async_copy(y_hbm.at[idx_s], rows, sem).wait()

            def edge(e, _):
                wrow = wv[e, pl.ds(0, 16)]
                w0 = wrow[col + 0]
                w1 = wrow[col + 1]
                w2 = wrow[col + 2]
                w3 = wrow[col + 3]
                for j in range(nj):
                    a = w0 * rows[e, pl.ds(0 * out_dim + j * 16, 16)]
                    a = a + w1 * rows[e, pl.ds(1 * out_dim + j * 16, 16)]
                    a = a + w2 * rows[e, pl.ds(2 * out_dim + j * 16, 16)]
                    a = a + w3 * rows[e, pl.ds(3 * out_dim + j * 16, 16)]
                    msg[e, pl.ds(j * 16, 16)] = a
                return _

            lax.fori_loop(0, C, edge, None)
            pltpu.sync_copy(msg, acc_sh.at[idx_d], add=True)
            return _

        lax.fori_loop(0, NCHUNK, chunk, None)
        plsc.subcore_barrier()
        pltpu.sync_copy(acc_sh.at[pl.ds(sid * RPT, RPT)],
                        part_hbm.at[cid, pl.ds(sid * RPT, RPT)])

    return body


@functools.cache
def _agg_kernel(out_dim, layer):
    ko = K * out_dim
    return pl.kernel(
        _make_agg_body(out_dim, layer),
        out_type=jax.ShapeDtypeStruct((NC, N, out_dim), jnp.float32),
        mesh=_mesh,
        compiler_params=_sc_params,
        scratch_types=[
            pltpu.VMEM((C,), jnp.int32),
            pltpu.VMEM((C,), jnp.int32),
            pltpu.VMEM((C, 16), jnp.float32),
            pltpu.VMEM((C, ko), jnp.float32),
            pltpu.VMEM((C, out_dim), jnp.float32),
            pltpu.VMEM((ZROWS, out_dim), jnp.float32),
            pltpu.VMEM_SHARED((N, out_dim), jnp.float32),
            pltpu.SemaphoreType.DMA,
        ],
    )


def _leaky(v):
    return jnp.where(v >= 0, v, 0.01 * v)


def _wcat(W):
    kin = W.shape[1]
    return jnp.transpose(W, (1, 0, 2)).reshape(kin, K * W.shape[2])


def kernel(x, edge_index, A1, Ab1, mu1, sigma1, W1, b1,
           A2, Ab2, mu2, sigma2, W2, b2, A3, Ab3, mu3, sigma3, W3, b3):
    src = edge_index[0]
    dst = edge_index[1]

    As = (A1, A2, A3)
    Abs = (Ab1, Ab2, Ab3)
    mus = (mu1, mu2, mu3)
    sigmas = (sigma1, sigma2, sigma3)
    Acat = jnp.concatenate(As, axis=1)

    m2 = jnp.zeros((48, 16), jnp.float32)
    m1 = jnp.zeros((48, 16), jnp.float32)
    cc = jnp.zeros((16,), jnp.float32)
    for l in range(3):
        s2 = sigmas[l] ** 2
        mp = mus[l] - Abs[l][None, :]
        for k in range(K):
            j = 4 * l + k
            m2 = m2.at[l * 16:(l + 1) * 16, j].set(-0.5 * s2[k])
            m1 = m1.at[l * 16:(l + 1) * 16, j].set(mp[k] * s2[k])
            cc = cc.at[j].set(-0.5 * jnp.sum(mp[k] ** 2 * s2[k]))
    cc = cc.reshape(1, 16)

    Wcat1 = _wcat(W1)
    Waug = jnp.zeros((128, 304), jnp.float32)
    Waug = Waug.at[:, :256].set(Wcat1).at[:3, 256:].set(Acat)
    y1p = _matmul(x, Waug, bn=400)
    y1 = y1p[:, :256]
    pcat = y1p[:, 256:]

    t = _diff_kernel(pcat, src, dst)

    w_all = _edge_weights(t, m2, m1, cc)

    p1 = _agg_kernel(64, 0)(y1, w_all, src, dst)
    x1 = _leaky((p1[0] + p1[1]) * (1.0 / K) + b1)
    xc = jnp.concatenate([x1, x], axis=1)

    y2 = _matmul(xc, _wcat(W2), bn=400)
    p2 = _agg_kernel(32, 1)(y2, w_all, src, dst)
    x2 = _leaky((p2[0] + p2[1]) * (1.0 / K) + b2)
    xc2 = jnp.concatenate([x2, xc], axis=1)

    y3 = _matmul(xc2, _wcat(W3), bn=400)
    p3 = _agg_kernel(16, 2)(y3, w_all, src, dst)
    x3 = _leaky((p3[0] + p3[1]) * (1.0 / K) + b3)
    return x3


# speedup vs baseline: 12.3291x; 1.9507x over previous
import functools

import jax
import jax.numpy as jnp
from jax import lax
from jax.experimental import pallas as pl
from jax.experimental.pallas import tpu as pltpu
from jax.experimental.pallas import tpu_sc as plsc

N = 10000
E = 320000
K = 4
EMB = 16

NC = 2
NS = 16
NW = NC * NS
EPW = E // NW
C = 80
NCHUNK = EPW // C
NLOOP = (NCHUNK - 1) // 2
RPT = N // NS
ZROWS = 125

assert NCHUNK == 2 * NLOOP + 1

_mesh = plsc.VectorSubcoreMesh(core_axis_name="c", subcore_axis_name="s")
_sc_params = pltpu.CompilerParams(use_tc_tiling_on_sc=False)


def _mm_body(x_ref, w_ref, o_ref):
    o_ref[...] = jnp.dot(x_ref[...], w_ref[...],
                         preferred_element_type=jnp.float32)


def _matmul(x, w, bn):
    n, kin = x.shape
    ko = w.shape[1]
    return pl.pallas_call(
        _mm_body,
        grid=(n // bn,),
        in_specs=[pl.BlockSpec((bn, kin), lambda i: (i, 0)),
                  pl.BlockSpec((kin, ko), lambda i: (0, 0))],
        out_specs=pl.BlockSpec((bn, ko), lambda i: (i, 0)),
        out_shape=jax.ShapeDtypeStruct((n, ko), jnp.float32),
    )(x, w)


def _wts_body(t_ref, m2_ref, m1_ref, c_ref, o_ref):
    t = t_ref[...]
    h = (jnp.dot(t * t, m2_ref[...], preferred_element_type=jnp.float32)
         + jnp.dot(t, m1_ref[...], preferred_element_type=jnp.float32)
         + c_ref[...])
    o_ref[...] = jnp.exp(h)


def _edge_weights(t, m2, m1, c, bn=2000):
    n = t.shape[0]
    return pl.pallas_call(
        _wts_body,
        grid=(n // bn,),
        in_specs=[pl.BlockSpec((bn, 48), lambda i: (i, 0)),
                  pl.BlockSpec((48, 16), lambda i: (0, 0)),
                  pl.BlockSpec((48, 16), lambda i: (0, 0)),
                  pl.BlockSpec((1, 16), lambda i: (0, 0))],
        out_specs=pl.BlockSpec((bn, 16), lambda i: (i, 0)),
        out_shape=jax.ShapeDtypeStruct((n, 16), jnp.float32),
    )(t, m2, m1, c)


def _diff_body(pcat_hbm, src_hbm, dst_hbm, out_hbm,
               src_v, dst_v, ps0, ps1, pd0, pd1, po0, po1,
               gsem0, gsem1, osem0, osem1):
    cid = lax.axis_index("c")
    sid = lax.axis_index("s")
    wid = cid * NS + sid
    base = wid * EPW
    row0 = wid * NCHUNK

    pltpu.sync_copy(src_hbm.at[pl.ds(row0, NCHUNK)], src_v)
    pltpu.sync_copy(dst_hbm.at[pl.ds(row0, NCHUNK)], dst_v)

    def issue_g(g, ps, pd, sem):
        pltpu.async_copy(pcat_hbm.at[src_v.at[g]], ps, sem)
        pltpu.async_copy(pcat_hbm.at[dst_v.at[g]], pd, sem)

    def wait_g(g, ps, pd, sem):
        pltpu.make_async_copy(pcat_hbm.at[src_v.at[g]], ps, sem).wait()
        pltpu.make_async_copy(pcat_hbm.at[dst_v.at[g]], pd, sem).wait()

    def issue_o(g, po, sem):
        pltpu.async_copy(po, out_hbm.at[pl.ds(base + g * C, C)], sem)

    def wait_o(g, po, sem):
        pltpu.make_async_copy(po, out_hbm.at[pl.ds(base + g * C, C)],
                              sem).wait()

    def compute(ps, pd, po):
        def edge(e, _):
            for j in range(3):
                sl = pl.ds(j * 16, 16)
                po[e, sl] = ps[e, sl] - pd[e, sl]
            return _
        lax.fori_loop(0, C, edge, None)

    issue_g(0, ps0, pd0, gsem0)

    def step(g2, _):
        a = 2 * g2
        b = a + 1
        issue_g(b, ps1, pd1, gsem1)
        wait_g(a, ps0, pd0, gsem0)

        @pl.when(g2 >= 1)
        def _w0():
            wait_o(a - 2, po0, osem0)

        compute(ps0, pd0, po0)
        issue_o(a, po0, osem0)
        issue_g(b + 1, ps0, pd0, gsem0)
        wait_g(b, ps1, pd1, gsem1)

        @pl.when(g2 >= 1)
        def _w1():
            wait_o(b - 2, po1, osem1)

        compute(ps1, pd1, po1)
        issue_o(b, po1, osem1)
        return _

    lax.fori_loop(0, NLOOP, step, None)
    gl = NCHUNK - 1
    wait_g(gl, ps0, pd0, gsem0)
    wait_o(gl - 2, po0, osem0)
    compute(ps0, pd0, po0)
    issue_o(gl, po0, osem0)
    wait_o(gl, po0, osem0)
    wait_o(gl - 1, po1, osem1)


_diff_kernel = pl.kernel(
    _diff_body,
    out_type=jax.ShapeDtypeStruct((E, 48), jnp.float32),
    mesh=_mesh,
    compiler_params=_sc_params,
    scratch_types=[
        pltpu.VMEM((NCHUNK, C), jnp.int32),
        pltpu.VMEM((NCHUNK, C), jnp.int32),
        pltpu.VMEM((C, 48), jnp.float32),
        pltpu.VMEM((C, 48), jnp.float32),
        pltpu.VMEM((C, 48), jnp.float32),
        pltpu.VMEM((C, 48), jnp.float32),
        pltpu.VMEM((C, 48), jnp.float32),
        pltpu.VMEM((C, 48), jnp.float32),
        pltpu.SemaphoreType.DMA,
        pltpu.SemaphoreType.DMA,
        pltpu.SemaphoreType.DMA,
        pltpu.SemaphoreType.DMA,
    ],
)


def _make_agg_body(out_dim, layer):
    ko = K * out_dim
    nj = out_dim // 16
    col = 4 * layer

    def body(y_hbm, w_hbm, src_hbm, dst_hbm, part_hbm,
             src_v, dst_v, wv0, wv1, rows0, rows1, msg0, msg1, zbuf, acc_sh,
             gsem0, gsem1, isem0, isem1, ssem0, ssem1):
        cid = lax.axis_index("c")
        sid = lax.axis_index("s")
        wid = cid * NS + sid
        base = wid * EPW
        row0 = wid * NCHUNK

        def zrow(i, _):
            for j in range(nj):
                zbuf[i, pl.ds(j * 16, 16)] = jnp.zeros((16,), jnp.float32)
            return _

        lax.fori_loop(0, ZROWS, zrow, None)
        for r in range(RPT // ZROWS):
            pltpu.sync_copy(zbuf,
                            acc_sh.at[pl.ds(sid * RPT + r * ZROWS, ZROWS)])
        plsc.subcore_barrier()

        pltpu.sync_copy(src_hbm.at[pl.ds(row0, NCHUNK)], src_v)
        pltpu.sync_copy(dst_hbm.at[pl.ds(row0, NCHUNK)], dst_v)

        def issue_g(g, rows, sem):
            pltpu.async_copy(y_hbm.at[src_v.at[g]], rows, sem)

        def wait_g(g, rows, sem):
            pltpu.make_async_copy(y_hbm.at[src_v.at[g]], rows, sem).wait()

        def issue_w(g, wv, sem):
            pltpu.async_copy(w_hbm.at[pl.ds(base + g * C, C)], wv, sem)

        def wait_w(g, wv, sem):
            pltpu.make_async_copy(w_hbm.at[pl.ds(base + g * C, C)], wv,
                                  sem).wait()

        def issue_s(g, msg, sem):
            pltpu.async_copy(msg, acc_sh.at[dst_v.at[g]], sem, add=True)

        def wait_s(g, msg, sem):
            pltpu.make_async_copy(msg, acc_sh.at[dst_v.at[g]], sem).wait()

        def compute(rows, wv, msg):
            def edge(e, _):
                wrow = wv[e, pl.ds(0, 16)]
                w0 = wrow[col + 0]
                w1 = wrow[col + 1]
                w2 = wrow[col + 2]
                w3 = wrow[col + 3]
                for j in range(nj):
                    a = w0 * rows[e, pl.ds(0 * out_dim + j * 16, 16)]
                    a = a + w1 * rows[e, pl.ds(1 * out_dim + j * 16, 16)]
                    a = a + w2 * rows[e, pl.ds(2 * out_dim + j * 16, 16)]
                    a = a + w3 * rows[e, pl.ds(3 * out_dim + j * 16, 16)]
                    msg[e, pl.ds(j * 16, 16)] = a
                return _
            lax.fori_loop(0, C, edge, None)

        issue_g(0, rows0, gsem0)
        issue_w(0, wv0, isem0)
        issue_w(1, wv1, isem1)

        def step(g2, _):
            a = 2 * g2
            b = a + 1
            issue_g(b, rows1, gsem1)
            wait_g(a, rows0, gsem0)
            wait_w(a, wv0, isem0)

            @pl.when(g2 >= 1)
            def _w0():
                wait_s(a - 2, msg0, ssem0)

            compute(rows0, wv0, msg0)
            issue_w(a + 2, wv0, isem0)
            issue_s(a, msg0, ssem0)
            issue_g(b + 1, rows0, gsem0)
            wait_g(b, rows1, gsem1)
            wait_w(b, wv1, isem1)

            @pl.when(g2 >= 1)
            def _w1():
                wait_s(b - 2, msg1, ssem1)

            compute(rows1, wv1, msg1)

            @pl.when(g2 < NLOOP - 1)
            def _iw():
                issue_w(b + 2, wv1, isem1)

            issue_s(b, msg1, ssem1)
            return _

        lax.fori_loop(0, NLOOP, step, None)
        gl = NCHUNK - 1
        wait_g(gl, rows0, gsem0)
        wait_w(gl, wv0, isem0)
        wait_s(gl - 2, msg0, ssem0)
        compute(rows0, wv0, msg0)
        issue_s(gl, msg0, ssem0)
        wait_s(gl, msg0, ssem0)
        wait_s(gl - 1, msg1, ssem1)

        plsc.subcore_barrier()
        pltpu.sync_copy(acc_sh.at[pl.ds(sid * RPT, RPT)],
                        part_hbm.at[cid, pl.ds(sid * RPT, RPT)])

    return body


@functools.cache
def _agg_kernel(out_dim, layer):
    ko = K * out_dim
    return pl.kernel(
        _make_agg_body(out_dim, layer),
        out_type=jax.ShapeDtypeStruct((NC, N, out_dim), jnp.float32),
        mesh=_mesh,
        compiler_params=_sc_params,
        scratch_types=[
            pltpu.VMEM((NCHUNK, C), jnp.int32),
            pltpu.VMEM((NCHUNK, C), jnp.int32),
            pltpu.VMEM((C, 16), jnp.float32),
            pltpu.VMEM((C, 16), jnp.float32),
            pltpu.VMEM((C, ko), jnp.float32),
            pltpu.VMEM((C, ko), jnp.float32),
            pltpu.VMEM((C, out_dim), jnp.float32),
            pltpu.VMEM((C, out_dim), jnp.float32),
            pltpu.VMEM((ZROWS, out_dim), jnp.float32),
            pltpu.VMEM_SHARED((N, out_dim), jnp.float32),
            pltpu.SemaphoreType.DMA,
            pltpu.SemaphoreType.DMA,
            pltpu.SemaphoreType.DMA,
            pltpu.SemaphoreType.DMA,
            pltpu.SemaphoreType.DMA,
            pltpu.SemaphoreType.DMA,
        ],
    )


def _leaky(v):
    return jnp.where(v >= 0, v, 0.01 * v)


def _wcat(W):
    kin = W.shape[1]
    return jnp.transpose(W, (1, 0, 2)).reshape(kin, K * W.shape[2])


def kernel(x, edge_index, A1, Ab1, mu1, sigma1, W1, b1,
           A2, Ab2, mu2, sigma2, W2, b2, A3, Ab3, mu3, sigma3, W3, b3):
    src = edge_index[0].reshape(E // C, C)
    dst = edge_index[1].reshape(E // C, C)

    As = (A1, A2, A3)
    Abs = (Ab1, Ab2, Ab3)
    mus = (mu1, mu2, mu3)
    sigmas = (sigma1, sigma2, sigma3)
    Acat = jnp.concatenate(As, axis=1)

    m2 = jnp.zeros((48, 16), jnp.float32)
    m1 = jnp.zeros((48, 16), jnp.float32)
    cc = jnp.zeros((16,), jnp.float32)
    for l in range(3):
        s2 = sigmas[l] ** 2
        mp = mus[l] - Abs[l][None, :]
        for k in range(K):
            j = 4 * l + k
            m2 = m2.at[l * 16:(l + 1) * 16, j].set(-0.5 * s2[k])
            m1 = m1.at[l * 16:(l + 1) * 16, j].set(mp[k] * s2[k])
            cc = cc.at[j].set(-0.5 * jnp.sum(mp[k] ** 2 * s2[k]))
    cc = cc.reshape(1, 16)

    Wcat1 = _wcat(W1)
    Waug = jnp.zeros((128, 304), jnp.float32)
    Waug = Waug.at[:, :256].set(Wcat1).at[:3, 256:].set(Acat)
    y1p = _matmul(x, Waug, bn=400)
    y1 = y1p[:, :256]
    pcat = y1p[:, 256:]

    t = _diff_kernel(pcat, src, dst)

    w_all = _edge_weights(t, m2, m1, cc)

    p1 = _agg_kernel(64, 0)(y1, w_all, src, dst)
    x1 = _leaky((p1[0] + p1[1]) * (1.0 / K) + b1)
    xc = jnp.concatenate([x1, x], axis=1)

    y2 = _matmul(xc, _wcat(W2), bn=400)
    p2 = _agg_kernel(32, 1)(y2, w_all, src, dst)
    x2 = _leaky((p2[0] + p2[1]) * (1.0 / K) + b2)
    xc2 = jnp.concatenate([x2, xc], axis=1)

    y3 = _matmul(xc2, _wcat(W3), bn=400)
    p3 = _agg_kernel(16, 2)(y3, w_all, src, dst)
    x3 = _leaky((p3[0] + p3[1]) * (1.0 / K) + b3)
    return x3
